# Initial kernel scaffold; baseline (speedup 1.0000x reference)
import jax
import jax.numpy as jnp
from jax.experimental import pallas as pl


def kernel(x, norm, edge_index):
    raise NotImplementedError("write your pallas kernel here")


# speedup vs baseline: 3.2460x; 3.2460x over previous
import functools

import jax
import jax.numpy as jnp
from jax import lax
from jax.experimental import pallas as pl
from jax.experimental.pallas import tpu as pltpu
from jax.experimental.pallas import tpu_sc as plsc

N = 10000
E = 320000
D = 128

NC = 2
NS = 16
NW = NC * NS
CPT = D // NW
CHUNK = 4000
NCHUNK = E // CHUNK
GROUPS = CHUNK // 16

NEG = float("-inf")


def _sc_body(xt_hbm, norm_hbm, pk_hbm, out_hbm, x_v, norm_v, acc_v, ebuf):
    wid = lax.axis_index("s") * NC + lax.axis_index("c")

    pltpu.sync_copy(xt_hbm.at[wid], x_v)
    pltpu.sync_copy(norm_hbm, norm_v)

    def init_body(j, _):
        acc_v[pl.ds(j * 16, 16)] = jnp.full((16,), NEG, jnp.float32)
        return _

    lax.fori_loop(0, N * CPT // 16, init_body, None)

    def group_body(g, _):
        p = ebuf[pl.ds(g * 16, 16)]
        src = p & 16383
        dst = lax.shift_right_logical(p, 14)
        ns = plsc.load_gather(norm_v, [src])
        nd = plsc.load_gather(norm_v, [dst])
        w = ns * nd
        s4 = src * CPT
        d4 = dst * CPT
        vs = []
        for c in range(CPT):
            xv = plsc.load_gather(x_v, [s4 + c])
            v = xv * w
            vs.append(v)
            old = plsc.load_gather(acc_v, [d4 + c])
            plsc.store_scatter(acc_v, [d4 + c], jnp.maximum(old, v))

        def fix_body(go):
            fails = None
            for c in range(CPT):
                cur = plsc.load_gather(acc_v, [d4 + c])
                m = vs[c] > cur
                plsc.store_scatter(acc_v, [d4 + c], vs[c], mask=m)
                fails = m if fails is None else (fails | m)
            return jnp.any(fails)

        lax.while_loop(lambda go: go, fix_body, jnp.bool_(True))
        return _

    def chunk_body(ci, _):
        pltpu.sync_copy(pk_hbm.at[pl.ds(ci * CHUNK, CHUNK)], ebuf)
        lax.fori_loop(0, GROUPS, group_body, None)
        return _

    lax.fori_loop(0, NCHUNK, chunk_body, None)

    def out_body(j, _):
        v = acc_v[pl.ds(j * 16, 16)]
        acc_v[pl.ds(j * 16, 16)] = jnp.where(v == NEG, jnp.float32(0.0), v)
        return _

    lax.fori_loop(0, N * CPT // 16, out_body, None)
    pltpu.sync_copy(acc_v, out_hbm.at[wid])


@functools.partial(
    pl.kernel,
    out_type=jax.ShapeDtypeStruct((NW, N * CPT), jnp.float32),
    mesh=plsc.VectorSubcoreMesh(core_axis_name="c", subcore_axis_name="s"),
    compiler_params=pltpu.CompilerParams(needs_layout_passes=False),
    scratch_types=[
        pltpu.VMEM((N * CPT,), jnp.float32),
        pltpu.VMEM((N,), jnp.float32),
        pltpu.VMEM((N * CPT,), jnp.float32),
        pltpu.VMEM((CHUNK,), jnp.int32),
    ],
)
def _sc_kernel(xt_hbm, norm_hbm, pk_hbm, out_hbm, x_v, norm_v, acc_v, ebuf):
    _sc_body(xt_hbm, norm_hbm, pk_hbm, out_hbm, x_v, norm_v, acc_v, ebuf)


def kernel(x, norm, edge_index):
    xt = x.reshape(N, NW, CPT).transpose(1, 0, 2).reshape(NW, N * CPT)
    ei = edge_index.astype(jnp.int32)
    packed = (ei[1] << 14) | ei[0]
    out = _sc_kernel(xt, norm.reshape(N), packed)
    return out.reshape(NW, N, CPT).transpose(1, 0, 2).reshape(N, D)


# speedup vs baseline: 8.2721x; 2.5484x over previous
import functools

import jax
import jax.numpy as jnp
from jax import lax
from jax.experimental import pallas as pl
from jax.experimental.pallas import tpu as pltpu
from jax.experimental.pallas import tpu_sc as plsc

N = 10000
E = 320000
D = 128

NC = 2
NS = 16
NW = NC * NS
CPT = D // NW
CHUNK = 4000
NCHUNK = E // CHUNK
GROUPS = CHUNK // 16
UNROLL = 5

NEG = float("-inf")


def _sc_body(xt_hbm, norm_hbm, pk_hbm, out_hbm, x_v, norm_v, acc_v, ebuf, sem):
    wid = lax.axis_index("s") * NC + lax.axis_index("c")

    pltpu.sync_copy(xt_hbm.at[wid], x_v)
    pltpu.sync_copy(norm_hbm, norm_v)

    def init_body(j, _):
        acc_v[pl.ds(j * 16, 16)] = jnp.full((16,), NEG, jnp.float32)
        return _

    lax.fori_loop(0, N * CPT // 16, init_body, None)

    def one_group(base):
        p = ebuf[pl.ds(base, 16)]
        src = p & 16383
        dst = lax.shift_right_logical(p, 14)
        ns = plsc.load_gather(norm_v, [src])
        nd = plsc.load_gather(norm_v, [dst])
        w = ns * nd
        s4 = src * CPT
        d4 = dst * CPT
        vks, dks = [], []
        for c in range(CPT):
            xv = plsc.load_gather(x_v, [s4 + c])
            vk, dk = plsc.sort_key_val(xv * w, d4 + c)
            vks.append(vk)
            dks.append(dk)
        olds = [plsc.load_gather(acc_v, [dks[c]]) for c in range(CPT)]
        for c in range(CPT):
            plsc.store_scatter(acc_v, [dks[c]], jnp.maximum(olds[c], vks[c]))

    def chunk_body(ci, _):
        slot = (ci & 1) * CHUNK
        pltpu.make_async_copy(
            pk_hbm.at[pl.ds(ci * CHUNK, CHUNK)],
            ebuf.at[pl.ds(slot, CHUNK)],
            sem,
        ).wait()

        nxt = ci + 1

        @pl.when(nxt < NCHUNK)
        def _start_next():
            pltpu.async_copy(
                pk_hbm.at[pl.ds(nxt * CHUNK, CHUNK)],
                ebuf.at[pl.ds((nxt & 1) * CHUNK, CHUNK)],
                sem,
            )

        def group_body(g, _):
            for u in range(UNROLL):
                one_group(slot + (g * UNROLL + u) * 16)
            return _

        lax.fori_loop(0, GROUPS // UNROLL, group_body, None)
        return _

    pltpu.async_copy(pk_hbm.at[pl.ds(0, CHUNK)], ebuf.at[pl.ds(0, CHUNK)], sem)
    lax.fori_loop(0, NCHUNK, chunk_body, None)

    def out_body(j, _):
        v = acc_v[pl.ds(j * 16, 16)]
        acc_v[pl.ds(j * 16, 16)] = jnp.where(v == NEG, jnp.float32(0.0), v)
        return _

    lax.fori_loop(0, N * CPT // 16, out_body, None)
    pltpu.sync_copy(acc_v, out_hbm.at[wid])


@functools.partial(
    pl.kernel,
    out_type=jax.ShapeDtypeStruct((NW, N * CPT), jnp.float32),
    mesh=plsc.VectorSubcoreMesh(core_axis_name="c", subcore_axis_name="s"),
    compiler_params=pltpu.CompilerParams(needs_layout_passes=False),
    scratch_types=[
        pltpu.VMEM((N * CPT,), jnp.float32),
        pltpu.VMEM((N,), jnp.float32),
        pltpu.VMEM((N * CPT,), jnp.float32),
        pltpu.VMEM((2 * CHUNK,), jnp.int32),
        pltpu.SemaphoreType.DMA,
    ],
)
def _sc_kernel(xt_hbm, norm_hbm, pk_hbm, out_hbm, x_v, norm_v, acc_v, ebuf, sem):
    _sc_body(xt_hbm, norm_hbm, pk_hbm, out_hbm, x_v, norm_v, acc_v, ebuf, sem)


def kernel(x, norm, edge_index):
    xt = x.reshape(N, NW, CPT).transpose(1, 0, 2).reshape(NW, N * CPT)
    ei = edge_index.astype(jnp.int32)
    packed = (ei[1] << 14) | ei[0]
    out = _sc_kernel(xt, norm.reshape(N), packed)
    return out.reshape(NW, N, CPT).transpose(1, 0, 2).reshape(N, D)


# speedup vs baseline: 15.1911x; 1.8364x over previous
import functools

import jax
import jax.numpy as jnp
from jax import lax
from jax.experimental import pallas as pl
from jax.experimental.pallas import tpu as pltpu
from jax.experimental.pallas import tpu_sc as plsc

N = 10000
E = 320000
D = 128

NC = 2
NS = 16
NW = NC * NS
CPT = D // NW
CHUNK = 4000
NCHUNK = E // CHUNK
GROUPS = CHUNK // 16
UNROLL = 5

NEG = float("-inf")


def _sc_body(xt_hbm, norm_hbm, pk_hbm, out_hbm, x_v, norm_v, acc_v, ebuf, sem):
    wid = lax.axis_index("s") * NC + lax.axis_index("c")

    pltpu.sync_copy(xt_hbm.at[wid], x_v)
    pltpu.sync_copy(norm_hbm, norm_v)

    def init_body(j, _):
        acc_v[pl.ds(j * 16, 16)] = jnp.full((16,), NEG, jnp.float32)
        return _

    lax.fori_loop(0, N * CPT // 16, init_body, None)

    def group_front(base):
        p = ebuf[pl.ds(base, 16)]
        src = p & 16383
        dst = lax.shift_right_logical(p, 14)
        ns = plsc.load_gather(norm_v, [src])
        nd = plsc.load_gather(norm_v, [dst])
        w = ns * nd
        s4 = src * CPT
        d4 = dst * CPT
        vks, dks = [], []
        for c in range(CPT):
            xv = plsc.load_gather(x_v, [s4 + c])
            vk, dk = plsc.sort_key_val(xv * w, d4 + c)
            vks.append(vk)
            dks.append(dk)
        return vks, dks

    def group_back(vks, dks):
        olds = [plsc.load_gather(acc_v, [dks[c]]) for c in range(CPT)]
        for c in range(CPT):
            plsc.store_scatter(acc_v, [dks[c]], jnp.maximum(olds[c], vks[c]))

    def chunk_body(ci, _):
        slot = (ci & 1) * CHUNK
        pltpu.make_async_copy(
            pk_hbm.at[pl.ds(ci * CHUNK, CHUNK)],
            ebuf.at[pl.ds(slot, CHUNK)],
            sem,
        ).wait()

        nxt = ci + 1

        @pl.when(nxt < NCHUNK)
        def _start_next():
            pltpu.async_copy(
                pk_hbm.at[pl.ds(nxt * CHUNK, CHUNK)],
                ebuf.at[pl.ds((nxt & 1) * CHUNK, CHUNK)],
                sem,
            )

        def group_body(g, _):
            fronts = [
                group_front(slot + (g * UNROLL + u) * 16) for u in range(UNROLL)
            ]
            for vks, dks in fronts:
                group_back(vks, dks)
            return _

        lax.fori_loop(0, GROUPS // UNROLL, group_body, None)
        return _

    pltpu.async_copy(pk_hbm.at[pl.ds(0, CHUNK)], ebuf.at[pl.ds(0, CHUNK)], sem)
    lax.fori_loop(0, NCHUNK, chunk_body, None)

    def out_body(j, _):
        v = acc_v[pl.ds(j * 16, 16)]
        acc_v[pl.ds(j * 16, 16)] = jnp.where(v == NEG, jnp.float32(0.0), v)
        return _

    lax.fori_loop(0, N * CPT // 16, out_body, None)
    pltpu.sync_copy(acc_v, out_hbm.at[wid])


@functools.partial(
    pl.kernel,
    out_type=jax.ShapeDtypeStruct((NW, N * CPT), jnp.float32),
    mesh=plsc.VectorSubcoreMesh(core_axis_name="c", subcore_axis_name="s"),
    compiler_params=pltpu.CompilerParams(needs_layout_passes=False),
    scratch_types=[
        pltpu.VMEM((N * CPT,), jnp.float32),
        pltpu.VMEM((N,), jnp.float32),
        pltpu.VMEM((N * CPT,), jnp.float32),
        pltpu.VMEM((2 * CHUNK,), jnp.int32),
        pltpu.SemaphoreType.DMA,
    ],
)
def _sc_kernel(xt_hbm, norm_hbm, pk_hbm, out_hbm, x_v, norm_v, acc_v, ebuf, sem):
    _sc_body(xt_hbm, norm_hbm, pk_hbm, out_hbm, x_v, norm_v, acc_v, ebuf, sem)


def kernel(x, norm, edge_index):
    xt = x.reshape(N, NW, CPT).transpose(1, 0, 2).reshape(NW, N * CPT)
    ei = edge_index.astype(jnp.int32)
    packed = (ei[1] << 14) | ei[0]
    out = _sc_kernel(xt, norm.reshape(N), packed)
    return out.reshape(NW, N, CPT).transpose(1, 0, 2).reshape(N, D)


# speedup vs baseline: 15.5880x; 1.0261x over previous
import functools

import jax
import jax.numpy as jnp
from jax import lax
from jax.experimental import pallas as pl
from jax.experimental.pallas import tpu as pltpu
from jax.experimental.pallas import tpu_sc as plsc

N = 10000
E = 320000
D = 128

NC = 2
NS = 16
NW = NC * NS
CPT = D // NW
CHUNK = 4000
NCHUNK = E // CHUNK
GROUPS = CHUNK // 16
UNROLL = 5

NEG = float("-inf")


def _sc_body(xt_hbm, norm_hbm, pk_hbm, out_hbm, x_v, norm_v, acc_v, ebuf, sem):
    wid = lax.axis_index("s") * NC + lax.axis_index("c")
    iot = lax.iota(jnp.int32, 16)

    pltpu.sync_copy(xt_hbm.at[wid], x_v)
    pltpu.sync_copy(norm_hbm, norm_v)

    def init_body(j, _):
        base = j * 16
        nv = plsc.load_gather(norm_v, [lax.shift_right_logical(base + iot, 2)])
        x_v[pl.ds(base, 16)] = x_v[pl.ds(base, 16)] * nv
        acc_v[pl.ds(base, 16)] = jnp.full((16,), NEG, jnp.float32)
        return _

    lax.fori_loop(0, N * CPT // 16, init_body, None)

    def group_front(base):
        p = ebuf[pl.ds(base, 16)]
        src = p & 16383
        dst = lax.shift_right_logical(p, 14)
        s4 = src * CPT
        d4 = dst * CPT
        vks, dks = [], []
        for c in range(CPT):
            xv = plsc.load_gather(x_v, [s4 + c])
            vk, dk = plsc.sort_key_val(xv, d4 + c)
            vks.append(vk)
            dks.append(dk)
        return vks, dks

    def group_back(vks, dks):
        olds = [plsc.load_gather(acc_v, [dks[c]]) for c in range(CPT)]
        for c in range(CPT):
            plsc.store_scatter(acc_v, [dks[c]], jnp.maximum(olds[c], vks[c]))

    def chunk_body(ci, _):
        slot = (ci & 1) * CHUNK
        pltpu.make_async_copy(
            pk_hbm.at[pl.ds(ci * CHUNK, CHUNK)],
            ebuf.at[pl.ds(slot, CHUNK)],
            sem,
        ).wait()

        nxt = ci + 1

        @pl.when(nxt < NCHUNK)
        def _start_next():
            pltpu.async_copy(
                pk_hbm.at[pl.ds(nxt * CHUNK, CHUNK)],
                ebuf.at[pl.ds((nxt & 1) * CHUNK, CHUNK)],
                sem,
            )

        def group_body(g, _):
            fronts = [
                group_front(slot + (g * UNROLL + u) * 16) for u in range(UNROLL)
            ]
            for vks, dks in fronts:
                group_back(vks, dks)
            return _

        lax.fori_loop(0, GROUPS // UNROLL, group_body, None)
        return _

    pltpu.async_copy(pk_hbm.at[pl.ds(0, CHUNK)], ebuf.at[pl.ds(0, CHUNK)], sem)
    lax.fori_loop(0, NCHUNK, chunk_body, None)

    def out_body(j, _):
        base = j * 16
        v = acc_v[pl.ds(base, 16)]
        nv = plsc.load_gather(norm_v, [lax.shift_right_logical(base + iot, 2)])
        acc_v[pl.ds(base, 16)] = jnp.where(v == NEG, jnp.float32(0.0), v * nv)
        return _

    lax.fori_loop(0, N * CPT // 16, out_body, None)
    pltpu.sync_copy(acc_v, out_hbm.at[wid])


@functools.partial(
    pl.kernel,
    out_type=jax.ShapeDtypeStruct((NW, N * CPT), jnp.float32),
    mesh=plsc.VectorSubcoreMesh(core_axis_name="c", subcore_axis_name="s"),
    compiler_params=pltpu.CompilerParams(needs_layout_passes=False),
    scratch_types=[
        pltpu.VMEM((N * CPT,), jnp.float32),
        pltpu.VMEM((N,), jnp.float32),
        pltpu.VMEM((N * CPT,), jnp.float32),
        pltpu.VMEM((2 * CHUNK,), jnp.int32),
        pltpu.SemaphoreType.DMA,
    ],
)
def _sc_kernel(xt_hbm, norm_hbm, pk_hbm, out_hbm, x_v, norm_v, acc_v, ebuf, sem):
    _sc_body(xt_hbm, norm_hbm, pk_hbm, out_hbm, x_v, norm_v, acc_v, ebuf, sem)


def kernel(x, norm, edge_index):
    xt = x.reshape(N, NW, CPT).transpose(1, 0, 2).reshape(NW, N * CPT)
    ei = edge_index.astype(jnp.int32)
    packed = (ei[1] << 14) | ei[0]
    out = _sc_kernel(xt, norm.reshape(N), packed)
    return out.reshape(NW, N, CPT).transpose(1, 0, 2).reshape(N, D)


# speedup vs baseline: 18.3096x; 1.1746x over previous
import functools

import jax
import jax.numpy as jnp
from jax import lax
from jax.experimental import pallas as pl
from jax.experimental.pallas import tpu as pltpu
from jax.experimental.pallas import tpu_sc as plsc

N = 10000
E = 320000
D = 128

NC = 2
NS = 16
NW = NC * NS
CPT = D // NW
CHUNK = 4000
NCHUNK = E // CHUNK
GROUPS = CHUNK // 16
UNROLL = 5

NEG = float("-inf")


def _sc_body(xt_hbm, norm_hbm, pk_hbm, out_hbm, refs):
    x_cs = refs[0:CPT]
    acc_cs = refs[CPT:2 * CPT]
    norm_v, ebuf, sem = refs[2 * CPT:]
    wid = lax.axis_index("s") * NC + lax.axis_index("c")

    for c in range(CPT):
        pltpu.sync_copy(xt_hbm.at[wid, c], x_cs[c])
    pltpu.sync_copy(norm_hbm, norm_v)

    def init_body(j, _):
        ds = pl.ds(j * 16, 16)
        nv = norm_v[ds]
        ninf = jnp.full((16,), NEG, jnp.float32)
        for c in range(CPT):
            x_cs[c][ds] = x_cs[c][ds] * nv
            acc_cs[c][ds] = ninf
        return _

    lax.fori_loop(0, N // 16, init_body, None)

    def group_front(base):
        p = ebuf[pl.ds(base, 16)]
        src = p & 16383
        dst = lax.shift_right_logical(p, 14)
        vks, dks = [], []
        for c in range(CPT):
            xv = plsc.load_gather(x_cs[c], [src])
            vk, dk = plsc.sort_key_val(xv, dst)
            vks.append(vk)
            dks.append(dk)
        return vks, dks

    def group_back(vks, dks):
        olds = [plsc.load_gather(acc_cs[c], [dks[c]]) for c in range(CPT)]
        for c in range(CPT):
            plsc.store_scatter(acc_cs[c], [dks[c]], jnp.maximum(olds[c], vks[c]))

    def chunk_body(ci, _):
        slot = (ci & 1) * CHUNK
        pltpu.make_async_copy(
            pk_hbm.at[pl.ds(ci * CHUNK, CHUNK)],
            ebuf.at[pl.ds(slot, CHUNK)],
            sem,
        ).wait()

        nxt = ci + 1

        @pl.when(nxt < NCHUNK)
        def _start_next():
            pltpu.async_copy(
                pk_hbm.at[pl.ds(nxt * CHUNK, CHUNK)],
                ebuf.at[pl.ds((nxt & 1) * CHUNK, CHUNK)],
                sem,
            )

        def group_body(g, _):
            fronts = [
                group_front(slot + (g * UNROLL + u) * 16) for u in range(UNROLL)
            ]
            for vks, dks in fronts:
                group_back(vks, dks)
            return _

        lax.fori_loop(0, GROUPS // UNROLL, group_body, None)
        return _

    pltpu.async_copy(pk_hbm.at[pl.ds(0, CHUNK)], ebuf.at[pl.ds(0, CHUNK)], sem)
    lax.fori_loop(0, NCHUNK, chunk_body, None)

    def out_body(j, _):
        ds = pl.ds(j * 16, 16)
        nv = norm_v[ds]
        for c in range(CPT):
            v = acc_cs[c][ds]
            acc_cs[c][ds] = jnp.where(v == NEG, jnp.float32(0.0), v * nv)
        return _

    lax.fori_loop(0, N // 16, out_body, None)
    for c in range(CPT):
        pltpu.sync_copy(acc_cs[c], out_hbm.at[wid, c])


@functools.partial(
    pl.kernel,
    out_type=jax.ShapeDtypeStruct((NW, CPT, N), jnp.float32),
    mesh=plsc.VectorSubcoreMesh(core_axis_name="c", subcore_axis_name="s"),
    compiler_params=pltpu.CompilerParams(needs_layout_passes=False),
    scratch_types=(
        [pltpu.VMEM((N,), jnp.float32) for _ in range(2 * CPT)]
        + [
            pltpu.VMEM((N,), jnp.float32),
            pltpu.VMEM((2 * CHUNK,), jnp.int32),
            pltpu.SemaphoreType.DMA,
        ]
    ),
)
def _sc_kernel(xt_hbm, norm_hbm, pk_hbm, out_hbm, *refs):
    _sc_body(xt_hbm, norm_hbm, pk_hbm, out_hbm, refs)


def kernel(x, norm, edge_index):
    xt = x.reshape(N, NW, CPT).transpose(1, 2, 0)
    ei = edge_index.astype(jnp.int32)
    packed = (ei[1] << 14) | ei[0]
    out = _sc_kernel(xt, norm.reshape(N), packed)
    return out.transpose(2, 0, 1).reshape(N, D)


# speedup vs baseline: 20.2784x; 1.1075x over previous
import functools

import jax
import jax.numpy as jnp
from jax import lax
from jax.experimental import pallas as pl
from jax.experimental.pallas import tpu as pltpu
from jax.experimental.pallas import tpu_sc as plsc

N = 10000
E = 320000
D = 128

NC = 2
NS = 16
NW = NC * NS
CPT = D // NW
CHUNK = 4000
NCHUNK = E // CHUNK
GROUPS = CHUNK // 16
UNROLL = 10

NEG = float("-inf")


def _sc_body(xt_hbm, norm_hbm, pk_hbm, out_hbm, refs):
    x_cs = refs[0:CPT]
    acc_cs = refs[CPT:2 * CPT]
    norm_v, ebuf, sem = refs[2 * CPT:]
    wid = lax.axis_index("s") * NC + lax.axis_index("c")

    for c in range(CPT):
        pltpu.sync_copy(xt_hbm.at[wid, c], x_cs[c])
    pltpu.sync_copy(norm_hbm, norm_v)

    def init_body(j, _):
        ds = pl.ds(j * 16, 16)
        nv = norm_v[ds]
        ninf = jnp.full((16,), NEG, jnp.float32)
        for c in range(CPT):
            x_cs[c][ds] = x_cs[c][ds] * nv
            acc_cs[c][ds] = ninf
        return _

    lax.fori_loop(0, N // 16, init_body, None)

    def group_front(base):
        p = ebuf[pl.ds(base, 16)]
        src = p & 16383
        dst = lax.shift_right_logical(p, 14)
        vks, dks = [], []
        for c in range(CPT):
            xv = plsc.load_gather(x_cs[c], [src])
            vk, dk = plsc.sort_key_val(xv, dst)
            vks.append(vk)
            dks.append(dk)
        return vks, dks

    def group_back(vks, dks):
        olds = [plsc.load_gather(acc_cs[c], [dks[c]]) for c in range(CPT)]
        for c in range(CPT):
            plsc.store_scatter(acc_cs[c], [dks[c]], jnp.maximum(olds[c], vks[c]))

    def chunk_body(ci, _):
        slot = (ci & 1) * CHUNK
        pltpu.make_async_copy(
            pk_hbm.at[pl.ds(ci * CHUNK, CHUNK)],
            ebuf.at[pl.ds(slot, CHUNK)],
            sem,
        ).wait()

        nxt = ci + 1

        @pl.when(nxt < NCHUNK)
        def _start_next():
            pltpu.async_copy(
                pk_hbm.at[pl.ds(nxt * CHUNK, CHUNK)],
                ebuf.at[pl.ds((nxt & 1) * CHUNK, CHUNK)],
                sem,
            )

        def group_body(g, _):
            fronts = [
                group_front(slot + (g * UNROLL + u) * 16) for u in range(UNROLL)
            ]
            for vks, dks in fronts:
                group_back(vks, dks)
            return _

        lax.fori_loop(0, GROUPS // UNROLL, group_body, None)
        return _

    pltpu.async_copy(pk_hbm.at[pl.ds(0, CHUNK)], ebuf.at[pl.ds(0, CHUNK)], sem)
    lax.fori_loop(0, NCHUNK, chunk_body, None)

    def out_body(j, _):
        ds = pl.ds(j * 16, 16)
        nv = norm_v[ds]
        for c in range(CPT):
            v = acc_cs[c][ds]
            acc_cs[c][ds] = jnp.where(v == NEG, jnp.float32(0.0), v * nv)
        return _

    lax.fori_loop(0, N // 16, out_body, None)
    for c in range(CPT):
        pltpu.sync_copy(acc_cs[c], out_hbm.at[wid, c])


@functools.partial(
    pl.kernel,
    out_type=jax.ShapeDtypeStruct((NW, CPT, N), jnp.float32),
    mesh=plsc.VectorSubcoreMesh(core_axis_name="c", subcore_axis_name="s"),
    compiler_params=pltpu.CompilerParams(needs_layout_passes=False),
    scratch_types=(
        [pltpu.VMEM((N,), jnp.float32) for _ in range(2 * CPT)]
        + [
            pltpu.VMEM((N,), jnp.float32),
            pltpu.VMEM((2 * CHUNK,), jnp.int32),
            pltpu.SemaphoreType.DMA,
        ]
    ),
)
def _sc_kernel(xt_hbm, norm_hbm, pk_hbm, out_hbm, *refs):
    _sc_body(xt_hbm, norm_hbm, pk_hbm, out_hbm, refs)


def kernel(x, norm, edge_index):
    xt = x.reshape(N, NW, CPT).transpose(1, 2, 0)
    ei = edge_index.astype(jnp.int32)
    packed = (ei[1] << 14) | ei[0]
    out = _sc_kernel(xt, norm.reshape(N), packed)
    return out.transpose(2, 0, 1).reshape(N, D)
